# baseline (speedup 1.0000x reference)
import jax
import jax.numpy as jnp
import numpy as np
from jax.experimental import pallas as pl
from jax.experimental.pallas import tpu as pltpu

_B, _L, _HID = 16, 512, 256
_T_END = 10000.0
_M = 5
_SCALES = (10.0 ** np.arange(_M)).astype(np.float32)
_LN_SCALES = np.log(_SCALES).astype(np.float32)
_C = 128
_NCHUNK = _L // _C
_HB = _B // 2


def _scan_kernel(sl_ref, pads_ref, wxq_ref, whq_ref, bq_ref, wxr_ref, whr_ref,
                 br_ref, wxs_ref, whs_ref, bs_ref,
                 bef_ref, aft_ref, delta_ref,
                 hh_ref, dsc_ref, buf_bef, buf_aft, tl_bef, tl_aft,
                 sem_bef, sem_aft, sem_tl):
    g = pl.program_id(0)
    p = jax.lax.rem(g, 2)

    @pl.when(g == 0)
    def _prologue():
        pads = pads_ref[...]
        cols = jax.lax.broadcasted_iota(jnp.int32, (_B, _L + 1), 1)
        colsL = jax.lax.broadcasted_iota(jnp.int32, (_B, _L), 1)
        diffs = jnp.concatenate(
            [pads[:, 0:1], pads[:, 1:] - pads[:, :-1]], axis=1)
        diffs_ext = jnp.concatenate(
            [diffs, jnp.full((_B, 1), -1.0, jnp.float32)], axis=1)
        rows = jax.lax.broadcasted_iota(jnp.int32, (_B, 1), 0)
        sl = jnp.zeros((_B, 1), jnp.int32)
        for b in range(_B):
            sl = jnp.where(rows == b, sl_ref[b], sl)
        t_last = jnp.sum(
            jnp.where(colsL == sl - 1, pads, 0.0), axis=1, keepdims=True)
        delta = jnp.where(cols < sl, diffs_ext, -1.0)
        delta = jnp.where(cols == sl, _T_END - t_last, delta)
        delta_ref[...] = delta
        dshift = jnp.concatenate([delta[:, 0:1], delta[:, :_L]], axis=1)
        for j in range(_NCHUNK):
            dsc_ref[j] = dshift[:, j * _C:(j + 1) * _C]
        dsc_ref[_NCHUNK] = jnp.broadcast_to(delta[:, _L - 1:_L], (_B, _C))
        hh_ref[...] = jnp.zeros((_M * _B, _HID), jnp.float32)

    @pl.when(g >= 2)
    def _wait_slot():
        pltpu.make_async_copy(
            buf_bef.at[p], bef_ref.at[:, :, pl.ds(0, _C), :], sem_bef.at[p]
        ).wait()
        pltpu.make_async_copy(
            buf_aft.at[p], aft_ref.at[:, :, pl.ds(0, _C), :], sem_aft.at[p]
        ).wait()

    whq = whq_ref[...]
    whr = whr_ref[...]
    whs = whs_ref[...]
    wxq = wxq_ref[...]
    wxr = wxr_ref[...]
    wxs = wxs_ref[...]
    bq = bq_ref[...]
    br = br_ref[...]
    bs = bs_ref[...]

    dt_chunk = dsc_ref[g]
    valid_chunk = dt_chunk >= 0.0
    dtc_chunk = jnp.where(valid_chunk, dt_chunk, 0.0)
    decay_chunk = [jnp.exp(dtc_chunk * (-1.0 / _SCALES[m])) for m in range(_M)]

    hh = hh_ref[...]
    hm = [[hh[m * _B + hb * _HB:m * _B + (hb + 1) * _HB] for m in range(_M)]
          for hb in range(2)]

    def one_step(hb, hmh, valid, dtc, decs, upd_gate):
        dm = [hmh[m] * decs[m] for m in range(_M)]
        h = dm[0] + dm[1] + dm[2] + dm[3] + dm[4]

        ltr = dtc * wxr + jnp.dot(
            h, whr, preferred_element_type=jnp.float32) + br
        lts = dtc * wxs + jnp.dot(
            h, whs, preferred_element_type=jnp.float32) + bs

        a = [-(ltr - _LN_SCALES[m]) ** 2 for m in range(_M)]
        mx = jnp.maximum(jnp.maximum(jnp.maximum(a[0], a[1]),
                                     jnp.maximum(a[2], a[3])), a[4])
        e = [jnp.exp(a[m] - mx) for m in range(_M)]
        rinv = 1.0 / (e[0] + e[1] + e[2] + e[3] + e[4])
        h_ret = (e[0] * dm[0] + e[1] * dm[1] + e[2] * dm[2]
                 + e[3] * dm[3] + e[4] * dm[4]) * rinv

        q = jnp.tanh(dtc * wxq + jnp.dot(
            h_ret, whq, preferred_element_type=jnp.float32) + bq)

        a2 = [-(lts - _LN_SCALES[m]) ** 2 for m in range(_M)]
        mx2 = jnp.maximum(jnp.maximum(jnp.maximum(a2[0], a2[1]),
                                      jnp.maximum(a2[2], a2[3])), a2[4])
        e2 = [jnp.exp(a2[m] - mx2) for m in range(_M)]
        r2inv = 1.0 / (e2[0] + e2[1] + e2[2] + e2[3] + e2[4])

        upd = valid & upd_gate
        new = [jnp.where(upd, dm[m] + (e2[m] * r2inv) * (q - dm[m]), hmh[m])
               for m in range(_M)]
        return dm, new

    for k in range(_C):
        dms = [None, None]
        for hb in range(2):
            r0 = hb * _HB
            gate = jnp.logical_not((g == 0) & (k == 0))
            dms[hb], hm[hb] = one_step(
                hb, hm[hb],
                valid_chunk[r0:r0 + _HB, k:k + 1],
                dtc_chunk[r0:r0 + _HB, k:k + 1],
                [decay_chunk[m][r0:r0 + _HB, k:k + 1] for m in range(_M)],
                gate)
        for m in range(_M):
            buf_bef[p, m, :, k:k + 1, :] = jnp.concatenate(
                [dms[0][m], dms[1][m]], axis=0).reshape(_B, 1, _HID)
            buf_aft[p, m, :, k:k + 1, :] = jnp.concatenate(
                [hm[0][m], hm[1][m]], axis=0).reshape(_B, 1, _HID)

    hh_ref[...] = jnp.concatenate(
        [hm[hb][m] for m in range(_M) for hb in range(2)], axis=0)

    off = g * _C
    pltpu.make_async_copy(
        buf_bef.at[p], bef_ref.at[:, :, pl.ds(off, _C), :], sem_bef.at[p]
    ).start()
    pltpu.make_async_copy(
        buf_aft.at[p], aft_ref.at[:, :, pl.ds(off, _C), :], sem_aft.at[p]
    ).start()

    @pl.when(g == _NCHUNK - 1)
    def _tail():
        dtt = dsc_ref[_NCHUNK]
        validt = dtt >= 0.0
        dtct = jnp.where(validt, dtt, 0.0)
        for hb in range(2):
            r0 = hb * _HB
            dtc = dtct[r0:r0 + _HB, 0:1]
            decs = [jnp.exp(dtc * (-1.0 / _SCALES[m])) for m in range(_M)]
            dm, newm = one_step(
                hb, hm[hb], validt[r0:r0 + _HB, 0:1], dtc, decs,
                jnp.bool_(True))
            for m in range(_M):
                tl_bef[m, r0:r0 + _HB, 0, :] = dm[m]
                tl_aft[m, r0:r0 + _HB, 0, :] = newm[m]
        cb = pltpu.make_async_copy(
            tl_bef, bef_ref.at[:, :, pl.ds(_L, 1), :], sem_tl.at[0])
        ca = pltpu.make_async_copy(
            tl_aft, aft_ref.at[:, :, pl.ds(_L, 1), :], sem_tl.at[1])
        cb.start()
        ca.start()
        cb.wait()
        ca.wait()
        for slot in range(2):
            pltpu.make_async_copy(
                buf_bef.at[slot], bef_ref.at[:, :, pl.ds(0, _C), :],
                sem_bef.at[slot]).wait()
            pltpu.make_async_copy(
                buf_aft.at[slot], aft_ref.at[:, :, pl.ds(0, _C), :],
                sem_aft.at[slot]).wait()


def kernel(seq_pads, seq_lens, Wx_q, Wh_q, b_q, Wx_r, Wh_r, b_r, Wx_s, Wh_s, b_s):
    const_spec2 = lambda shape: pl.BlockSpec(shape, lambda g: (0, 0))
    row = lambda v: v.reshape(1, _HID)

    bef, aft, delta2d = pl.pallas_call(
        _scan_kernel,
        grid=(_NCHUNK,),
        in_specs=[
            pl.BlockSpec(memory_space=pltpu.MemorySpace.SMEM),
            const_spec2((_B, _L)),
            const_spec2((1, _HID)),
            const_spec2((_HID, _HID)),
            const_spec2((1, _HID)),
            const_spec2((1, _HID)),
            const_spec2((_HID, _HID)),
            const_spec2((1, _HID)),
            const_spec2((1, _HID)),
            const_spec2((_HID, _HID)),
            const_spec2((1, _HID)),
        ],
        out_specs=[
            pl.BlockSpec(memory_space=pl.ANY),
            pl.BlockSpec(memory_space=pl.ANY),
            const_spec2((_B, _L + 1)),
        ],
        out_shape=[
            jax.ShapeDtypeStruct((_M, _B, _L + 1, _HID), jnp.float32),
            jax.ShapeDtypeStruct((_M, _B, _L + 1, _HID), jnp.float32),
            jax.ShapeDtypeStruct((_B, _L + 1), jnp.float32),
        ],
        scratch_shapes=[
            pltpu.VMEM((_M * _B, _HID), jnp.float32),
            pltpu.VMEM((_NCHUNK + 1, _B, _C), jnp.float32),
            pltpu.VMEM((2, _M, _B, _C, _HID), jnp.float32),
            pltpu.VMEM((2, _M, _B, _C, _HID), jnp.float32),
            pltpu.VMEM((_M, _B, 1, _HID), jnp.float32),
            pltpu.VMEM((_M, _B, 1, _HID), jnp.float32),
            pltpu.SemaphoreType.DMA((2,)),
            pltpu.SemaphoreType.DMA((2,)),
            pltpu.SemaphoreType.DMA((2,)),
        ],
        compiler_params=pltpu.CompilerParams(
            dimension_semantics=("arbitrary",)),
    )(seq_lens, seq_pads.reshape(_B, _L), Wx_q, Wh_q, row(b_q), Wx_r, Wh_r,
      row(b_r), Wx_s, Wh_s, row(b_s))

    return bef, aft, delta2d[:, :, None]


# speedup vs baseline: 1.0385x; 1.0385x over previous
import jax
import jax.numpy as jnp
import numpy as np
from jax.experimental import pallas as pl
from jax.experimental.pallas import tpu as pltpu

_B, _L, _HID = 16, 512, 256
_T_END = 10000.0
_M = 5
_SCALES = (10.0 ** np.arange(_M)).astype(np.float32)
_LN_SCALES = np.log(_SCALES).astype(np.float32)
_C = 64
_NCHUNK = _L // _C
_HB = _B // 2


def _scan_kernel(sl_ref, pads_ref, wxq_ref, whq_ref, bq_ref, wxr_ref, whr_ref,
                 br_ref, wxs_ref, whs_ref, bs_ref,
                 bef_ref, aft_ref, delta_ref,
                 hh_ref, dsc_ref, buf_bef, buf_aft, tl_bef, tl_aft,
                 sem_bef, sem_aft, sem_tl):
    g = pl.program_id(0)
    p = jax.lax.rem(g, 2)

    @pl.when(g == 0)
    def _prologue():
        pads = pads_ref[...]
        cols = jax.lax.broadcasted_iota(jnp.int32, (_B, _L + 1), 1)
        colsL = jax.lax.broadcasted_iota(jnp.int32, (_B, _L), 1)
        diffs = jnp.concatenate(
            [pads[:, 0:1], pads[:, 1:] - pads[:, :-1]], axis=1)
        diffs_ext = jnp.concatenate(
            [diffs, jnp.full((_B, 1), -1.0, jnp.float32)], axis=1)
        rows = jax.lax.broadcasted_iota(jnp.int32, (_B, 1), 0)
        sl = jnp.zeros((_B, 1), jnp.int32)
        for b in range(_B):
            sl = jnp.where(rows == b, sl_ref[b], sl)
        t_last = jnp.sum(
            jnp.where(colsL == sl - 1, pads, 0.0), axis=1, keepdims=True)
        delta = jnp.where(cols < sl, diffs_ext, -1.0)
        delta = jnp.where(cols == sl, _T_END - t_last, delta)
        delta_ref[...] = delta
        dshift = jnp.concatenate([delta[:, 0:1], delta[:, :_L]], axis=1)
        for j in range(_NCHUNK):
            dsc_ref[j] = dshift[:, j * _C:(j + 1) * _C]
        dsc_ref[_NCHUNK] = jnp.broadcast_to(delta[:, _L - 1:_L], (_B, _C))
        hh_ref[...] = jnp.zeros((_M * _B, _HID), jnp.float32)

    @pl.when(g >= 2)
    def _wait_slot():
        pltpu.make_async_copy(
            buf_bef.at[p], bef_ref.at[:, :, pl.ds(0, _C), :], sem_bef.at[p]
        ).wait()
        pltpu.make_async_copy(
            buf_aft.at[p], aft_ref.at[:, :, pl.ds(0, _C), :], sem_aft.at[p]
        ).wait()

    whq = whq_ref[...]
    whr = whr_ref[...]
    whs = whs_ref[...]
    wxq = wxq_ref[...]
    wxr = wxr_ref[...]
    wxs = wxs_ref[...]
    bq = bq_ref[...]
    br = br_ref[...]
    bs = bs_ref[...]

    dt_chunk = dsc_ref[g]
    valid_chunk = dt_chunk >= 0.0
    dtc_chunk = jnp.where(valid_chunk, dt_chunk, 0.0)
    decay_chunk = [jnp.exp(dtc_chunk * (-1.0 / _SCALES[m])) for m in range(_M)]

    hh = hh_ref[...]
    hm = [[hh[m * _B + hb * _HB:m * _B + (hb + 1) * _HB] for m in range(_M)]
          for hb in range(2)]

    def one_step(hb, hmh, valid, dtc, decs, upd_gate):
        dm = [hmh[m] * decs[m] for m in range(_M)]
        h = dm[0] + dm[1] + dm[2] + dm[3] + dm[4]

        ltr = dtc * wxr + jnp.dot(
            h, whr, preferred_element_type=jnp.float32) + br
        lts = dtc * wxs + jnp.dot(
            h, whs, preferred_element_type=jnp.float32) + bs

        a = [-(ltr - _LN_SCALES[m]) ** 2 for m in range(_M)]
        mx = jnp.maximum(jnp.maximum(jnp.maximum(a[0], a[1]),
                                     jnp.maximum(a[2], a[3])), a[4])
        e = [jnp.exp(a[m] - mx) for m in range(_M)]
        rinv = 1.0 / (e[0] + e[1] + e[2] + e[3] + e[4])
        h_ret = (e[0] * dm[0] + e[1] * dm[1] + e[2] * dm[2]
                 + e[3] * dm[3] + e[4] * dm[4]) * rinv

        q = jnp.tanh(dtc * wxq + jnp.dot(
            h_ret, whq, preferred_element_type=jnp.float32) + bq)

        a2 = [-(lts - _LN_SCALES[m]) ** 2 for m in range(_M)]
        mx2 = jnp.maximum(jnp.maximum(jnp.maximum(a2[0], a2[1]),
                                      jnp.maximum(a2[2], a2[3])), a2[4])
        e2 = [jnp.exp(a2[m] - mx2) for m in range(_M)]
        r2inv = 1.0 / (e2[0] + e2[1] + e2[2] + e2[3] + e2[4])

        upd = valid & upd_gate
        new = [jnp.where(upd, dm[m] + (e2[m] * r2inv) * (q - dm[m]), hmh[m])
               for m in range(_M)]
        return dm, new

    for k in range(_C):
        for hb in range(2):
            r0 = hb * _HB
            gate = jnp.logical_not((g == 0) & (k == 0))
            dm, hm[hb] = one_step(
                hb, hm[hb],
                valid_chunk[r0:r0 + _HB, k:k + 1],
                dtc_chunk[r0:r0 + _HB, k:k + 1],
                [decay_chunk[m][r0:r0 + _HB, k:k + 1] for m in range(_M)],
                gate)
            for m in range(_M):
                buf_bef[p, m, r0:r0 + _HB, k:k + 1, :] = dm[m].reshape(
                    _HB, 1, _HID)
                buf_aft[p, m, r0:r0 + _HB, k:k + 1, :] = hm[hb][m].reshape(
                    _HB, 1, _HID)

    hh_ref[...] = jnp.concatenate(
        [hm[hb][m] for m in range(_M) for hb in range(2)], axis=0)

    off = g * _C
    pltpu.make_async_copy(
        buf_bef.at[p], bef_ref.at[:, :, pl.ds(off, _C), :], sem_bef.at[p]
    ).start()
    pltpu.make_async_copy(
        buf_aft.at[p], aft_ref.at[:, :, pl.ds(off, _C), :], sem_aft.at[p]
    ).start()

    @pl.when(g == _NCHUNK - 1)
    def _tail():
        dtt = dsc_ref[_NCHUNK]
        validt = dtt >= 0.0
        dtct = jnp.where(validt, dtt, 0.0)
        for hb in range(2):
            r0 = hb * _HB
            dtc = dtct[r0:r0 + _HB, 0:1]
            decs = [jnp.exp(dtc * (-1.0 / _SCALES[m])) for m in range(_M)]
            dm, newm = one_step(
                hb, hm[hb], validt[r0:r0 + _HB, 0:1], dtc, decs,
                jnp.bool_(True))
            for m in range(_M):
                tl_bef[m, r0:r0 + _HB, 0, :] = dm[m]
                tl_aft[m, r0:r0 + _HB, 0, :] = newm[m]
        cb = pltpu.make_async_copy(
            tl_bef, bef_ref.at[:, :, pl.ds(_L, 1), :], sem_tl.at[0])
        ca = pltpu.make_async_copy(
            tl_aft, aft_ref.at[:, :, pl.ds(_L, 1), :], sem_tl.at[1])
        cb.start()
        ca.start()
        cb.wait()
        ca.wait()
        for slot in range(2):
            pltpu.make_async_copy(
                buf_bef.at[slot], bef_ref.at[:, :, pl.ds(0, _C), :],
                sem_bef.at[slot]).wait()
            pltpu.make_async_copy(
                buf_aft.at[slot], aft_ref.at[:, :, pl.ds(0, _C), :],
                sem_aft.at[slot]).wait()


def kernel(seq_pads, seq_lens, Wx_q, Wh_q, b_q, Wx_r, Wh_r, b_r, Wx_s, Wh_s, b_s):
    const_spec2 = lambda shape: pl.BlockSpec(shape, lambda g: (0, 0))
    row = lambda v: v.reshape(1, _HID)

    bef, aft, delta2d = pl.pallas_call(
        _scan_kernel,
        grid=(_NCHUNK,),
        in_specs=[
            pl.BlockSpec(memory_space=pltpu.MemorySpace.SMEM),
            const_spec2((_B, _L)),
            const_spec2((1, _HID)),
            const_spec2((_HID, _HID)),
            const_spec2((1, _HID)),
            const_spec2((1, _HID)),
            const_spec2((_HID, _HID)),
            const_spec2((1, _HID)),
            const_spec2((1, _HID)),
            const_spec2((_HID, _HID)),
            const_spec2((1, _HID)),
        ],
        out_specs=[
            pl.BlockSpec(memory_space=pl.ANY),
            pl.BlockSpec(memory_space=pl.ANY),
            const_spec2((_B, _L + 1)),
        ],
        out_shape=[
            jax.ShapeDtypeStruct((_M, _B, _L + 1, _HID), jnp.float32),
            jax.ShapeDtypeStruct((_M, _B, _L + 1, _HID), jnp.float32),
            jax.ShapeDtypeStruct((_B, _L + 1), jnp.float32),
        ],
        scratch_shapes=[
            pltpu.VMEM((_M * _B, _HID), jnp.float32),
            pltpu.VMEM((_NCHUNK + 1, _B, _C), jnp.float32),
            pltpu.VMEM((2, _M, _B, _C, _HID), jnp.float32),
            pltpu.VMEM((2, _M, _B, _C, _HID), jnp.float32),
            pltpu.VMEM((_M, _B, 1, _HID), jnp.float32),
            pltpu.VMEM((_M, _B, 1, _HID), jnp.float32),
            pltpu.SemaphoreType.DMA((2,)),
            pltpu.SemaphoreType.DMA((2,)),
            pltpu.SemaphoreType.DMA((2,)),
        ],
        compiler_params=pltpu.CompilerParams(
            dimension_semantics=("arbitrary",)),
    )(seq_lens, seq_pads.reshape(_B, _L), Wx_q, Wh_q, row(b_q), Wx_r, Wh_r,
      row(b_r), Wx_s, Wh_s, row(b_s))

    return bef, aft, delta2d[:, :, None]
